# Initial kernel scaffold; baseline (speedup 1.0000x reference)
import jax
import jax.numpy as jnp
from jax.experimental import pallas as pl


def kernel(x, edge_index, W1_l, b1, W1_r, W2_l, b2, W2_r):
    raise NotImplementedError("write your pallas kernel here")


# speedup vs baseline: 7.2591x; 7.2591x over previous
import functools

import jax
import jax.numpy as jnp
from jax import lax
from jax.experimental import pallas as pl
from jax.experimental.pallas import tpu as pltpu
from jax.experimental.pallas import tpu_sc as plsc

N_NODES = 10000
N_PAD = 10240
ROWS_PER_TEC = 640
E = 320000
CHUNK = 80
N_TECS = 32
CHUNKS_PER_TEC = E // N_TECS // CHUNK
D = 128


def _sc_aggregate(src2d, dst2d, table):
    mesh = plsc.VectorSubcoreMesh(core_axis_name="c", subcore_axis_name="s")

    @functools.partial(
        pl.kernel,
        mesh=mesh,
        out_type=(
            jax.ShapeDtypeStruct((2, N_PAD, D), jnp.float32),
            jax.ShapeDtypeStruct((2, N_PAD), jnp.float32),
        ),
        scratch_types=[
            pltpu.VMEM((CHUNKS_PER_TEC, CHUNK), jnp.int32),
            pltpu.VMEM((CHUNKS_PER_TEC, CHUNK), jnp.int32),
            pltpu.VMEM((CHUNK, D), jnp.float32),
            pltpu.VMEM((16, D), jnp.float32),
            pltpu.VMEM((ROWS_PER_TEC,), jnp.float32),
            pltpu.VMEM((CHUNK,), jnp.float32),
            pltpu.VMEM_SHARED((N_PAD, D), jnp.float32),
            pltpu.VMEM_SHARED((N_PAD,), jnp.float32),
            pltpu.SemaphoreType.DMA,
        ],
    )
    def agg_kernel(src_hbm, dst_hbm, tab_hbm, agg_hbm, deg_hbm,
                   src_v, dst_v, rows_v, ztile_v, zdeg_v, ones_v,
                   acc_sh, dacc_sh, sem):
        c = lax.axis_index("c")
        s = lax.axis_index("s")

        zeros16 = jnp.zeros((16,), jnp.float32)
        ones16 = jnp.ones((16,), jnp.float32)
        for r in range(16):
            for k in range(D // 16):
                ztile_v[r, pl.ds(k * 16, 16)] = zeros16
        for k in range(CHUNK // 16):
            ones_v[pl.ds(k * 16, 16)] = ones16

        def _zdeg(i, carry):
            zdeg_v[pl.ds(i * 16, 16)] = zeros16
            return carry
        lax.fori_loop(0, ROWS_PER_TEC // 16, _zdeg, 0)

        row0 = s * ROWS_PER_TEC

        def _zacc(j, carry):
            pltpu.sync_copy(ztile_v, acc_sh.at[pl.ds(row0 + j * 16, 16)])
            return carry
        lax.fori_loop(0, ROWS_PER_TEC // 16, _zacc, 0)
        pltpu.sync_copy(zdeg_v, dacc_sh.at[pl.ds(row0, ROWS_PER_TEC)])

        plsc.subcore_barrier()

        wid = c * 16 + s
        pltpu.sync_copy(src_hbm.at[wid], src_v)
        pltpu.sync_copy(dst_hbm.at[wid], dst_v)

        def _edges(i, carry):
            pltpu.async_copy(tab_hbm.at[src_v.at[i]], rows_v, sem).wait()
            pltpu.sync_copy(rows_v, acc_sh.at[dst_v.at[i]], add=True)
            pltpu.sync_copy(ones_v, dacc_sh.at[dst_v.at[i]], add=True)
            return carry
        lax.fori_loop(0, CHUNKS_PER_TEC, _edges, 0)

        plsc.subcore_barrier()

        pltpu.sync_copy(acc_sh.at[pl.ds(row0, ROWS_PER_TEC)],
                        agg_hbm.at[c, pl.ds(row0, ROWS_PER_TEC)])
        pltpu.sync_copy(dacc_sh.at[pl.ds(row0, ROWS_PER_TEC)],
                        deg_hbm.at[c, pl.ds(row0, ROWS_PER_TEC)])

    return agg_kernel(src2d, dst2d, table)


def _tc_dense(agg, deg, xin, W_l, b, W_r, relu):
    B = 512

    def body(agg_ref, deg_ref, x_ref, wl_ref, wr_ref, b_ref, o_ref):
        ssum = agg_ref[0] + agg_ref[1]
        dsum = deg_ref[0] + deg_ref[1]
        mean = ssum / jnp.maximum(dsum, 1.0)
        acc = (jnp.dot(mean, wl_ref[...], preferred_element_type=jnp.float32)
               + jnp.dot(x_ref[...], wr_ref[...], preferred_element_type=jnp.float32)
               + b_ref[...])
        o_ref[...] = jnp.maximum(acc, 0.0) if relu else acc

    return pl.pallas_call(
        body,
        grid=(N_PAD // B,),
        in_specs=[
            pl.BlockSpec((2, B, D), lambda i: (0, i, 0)),
            pl.BlockSpec((2, B, 1), lambda i: (0, i, 0)),
            pl.BlockSpec((B, D), lambda i: (i, 0)),
            pl.BlockSpec((D, D), lambda i: (0, 0)),
            pl.BlockSpec((D, D), lambda i: (0, 0)),
            pl.BlockSpec((1, D), lambda i: (0, 0)),
        ],
        out_specs=pl.BlockSpec((B, D), lambda i: (i, 0)),
        out_shape=jax.ShapeDtypeStruct((N_PAD, D), jnp.float32),
    )(agg, deg, xin, W_l, W_r, b)


def kernel(x, edge_index, W1_l, b1, W1_r, W2_l, b2, W2_r):
    src = edge_index[0].astype(jnp.int32).reshape(N_TECS, CHUNKS_PER_TEC, CHUNK)
    dst = edge_index[1].astype(jnp.int32).reshape(N_TECS, CHUNKS_PER_TEC, CHUNK)
    x_pad = jnp.pad(x, ((0, N_PAD - N_NODES), (0, 0)))
    b1r = b1.reshape(1, D)
    b2r = b2.reshape(1, D)

    agg1, deg1 = _sc_aggregate(src, dst, x_pad)
    h = _tc_dense(agg1, deg1.reshape(2, N_PAD, 1), x_pad, W1_l, b1r, W1_r,
                  relu=True)
    agg2, deg2 = _sc_aggregate(src, dst, h)
    out = _tc_dense(agg2, deg2.reshape(2, N_PAD, 1), h, W2_l, b2r, W2_r,
                    relu=False)
    return out[:N_NODES]


# speedup vs baseline: 11.4526x; 1.5777x over previous
import functools

import jax
import jax.numpy as jnp
from jax import lax
from jax.experimental import pallas as pl
from jax.experimental.pallas import tpu as pltpu
from jax.experimental.pallas import tpu_sc as plsc

N_NODES = 10000
N_PAD = 10240
ROWS_PER_TEC = 640
E = 320000
CHUNK = 80
N_TECS = 32
CHUNKS_PER_TEC = E // N_TECS // CHUNK
PHASES = (64, 61)
IDX_ROWS = max(PHASES)
D = 128


def _sc_aggregate(src2d, dst2d, table, with_deg):
    mesh = plsc.VectorSubcoreMesh(core_axis_name="c", subcore_axis_name="s")

    out_type = [jax.ShapeDtypeStruct((2, N_PAD, D), jnp.float32)]
    scratch = [
        pltpu.VMEM((IDX_ROWS, CHUNK), jnp.int32),
        pltpu.VMEM((IDX_ROWS, CHUNK), jnp.int32),
        pltpu.VMEM((CHUNK, D), jnp.float32),
        pltpu.VMEM((CHUNK, D), jnp.float32),
        pltpu.VMEM((16, D), jnp.float32),
        pltpu.VMEM_SHARED((N_PAD, D), jnp.float32),
        pltpu.SemaphoreType.DMA,
        pltpu.SemaphoreType.DMA,
        pltpu.SemaphoreType.DMA,
        pltpu.SemaphoreType.DMA,
    ]
    if with_deg:
        out_type.append(jax.ShapeDtypeStruct((2, N_PAD), jnp.float32))
        scratch += [
            pltpu.VMEM((CHUNK,), jnp.float32),
            pltpu.VMEM_SHARED((N_PAD,), jnp.float32),
            pltpu.SemaphoreType.DMA,
            pltpu.SemaphoreType.DMA,
        ]

    @functools.partial(pl.kernel, mesh=mesh, out_type=tuple(out_type),
                       scratch_types=scratch)
    def agg_kernel(src_hbm, dst_hbm, tab_hbm, agg_hbm, *rest):
        if with_deg:
            (deg_hbm, src_v, dst_v, buf0, buf1, ztile_v, acc_sh,
             g0, g1, s0, s1, ones_v, dacc_sh, d0, d1) = rest
        else:
            (src_v, dst_v, buf0, buf1, ztile_v, acc_sh,
             g0, g1, s0, s1) = rest
        c = lax.axis_index("c")
        s = lax.axis_index("s")
        wid = c * 16 + s
        row0 = s * ROWS_PER_TEC
        bufs = (buf0, buf1)
        gsems = (g0, g1)
        ssems = (s0, s1)

        zeros16 = jnp.zeros((16,), jnp.float32)

        def _zt(r, carry):
            for k in range(D // 16):
                ztile_v[r, pl.ds(k * 16, 16)] = zeros16
            return carry
        lax.fori_loop(0, 16, _zt, 0)

        def _zacc(j, carry):
            pltpu.sync_copy(ztile_v, acc_sh.at[pl.ds(row0 + j * 16, 16)])
            return carry
        lax.fori_loop(0, ROWS_PER_TEC // 16, _zacc, 0)

        if with_deg:
            ones16 = jnp.ones((16,), jnp.float32)
            for k in range(CHUNK // 16):
                ones_v[pl.ds(k * 16, 16)] = ones16
            for k in range(ROWS_PER_TEC // D):
                pltpu.sync_copy(ztile_v.at[0],
                                dacc_sh.at[pl.ds(row0 + k * D, D)])

        plsc.subcore_barrier()

        def _gwait(b):
            pltpu.make_async_copy(tab_hbm.at[src_v.at[0]], bufs[b],
                                  gsems[b]).wait()

        def _swait(b):
            pltpu.make_async_copy(bufs[b], acc_sh.at[dst_v.at[0]],
                                  ssems[b]).wait()

        if with_deg:
            dsems = (d0, d1)

            def _dwait(b):
                pltpu.make_async_copy(ones_v, dacc_sh.at[dst_v.at[0]],
                                      dsems[b]).wait()

        def _deg_step(p, ci, b):
            pl.when(p > 0)(lambda: _dwait(b))
            pltpu.async_copy(ones_v, dacc_sh.at[dst_v.at[ci]], dsems[b],
                             add=True)

        def _phase(off, n):
            even = n % 2 == 0
            np_ = (n - 2) // 2 if even else (n - 1) // 2
            pltpu.sync_copy(src_hbm.at[wid, pl.ds(off, n)],
                            src_v.at[pl.ds(0, n)])
            pltpu.sync_copy(dst_hbm.at[wid, pl.ds(off, n)],
                            dst_v.at[pl.ds(0, n)])
            pltpu.async_copy(tab_hbm.at[src_v.at[0]], buf0, g0)
            pltpu.async_copy(tab_hbm.at[src_v.at[1]], buf1, g1)

            def _pair(p, carry):
                ca = 2 * p
                _gwait(0)
                pltpu.async_copy(bufs[0], acc_sh.at[dst_v.at[ca]], s0,
                                 add=True)
                if with_deg:
                    _deg_step(p, ca, 0)
                _gwait(1)
                _swait(0)
                nxt_e = jnp.minimum(ca + 2, n - 1)
                pltpu.async_copy(tab_hbm.at[src_v.at[nxt_e]], buf0, g0)
                pltpu.async_copy(bufs[1], acc_sh.at[dst_v.at[ca + 1]], s1,
                                 add=True)
                if with_deg:
                    _deg_step(p, ca + 1, 1)
                _swait(1)
                nxt_o = jnp.minimum(ca + 3, n - 1)

                def _prefetch_odd():
                    pltpu.async_copy(tab_hbm.at[src_v.at[nxt_o]], buf1, g1)
                if even:
                    _prefetch_odd()
                else:
                    pl.when(p < np_ - 1)(_prefetch_odd)
                return carry
            lax.fori_loop(0, np_, _pair, 0)

            _gwait(0)
            pltpu.async_copy(bufs[0], acc_sh.at[dst_v.at[n - 2 if even else n - 1]],
                             s0, add=True)
            if with_deg:
                _dwait(0)
                pltpu.sync_copy(ones_v,
                                dacc_sh.at[dst_v.at[n - 2 if even else n - 1]],
                                add=True)
            if even:
                _gwait(1)
                _swait(0)
                pltpu.async_copy(bufs[1], acc_sh.at[dst_v.at[n - 1]], s1,
                                 add=True)
                if with_deg:
                    _dwait(1)
                    pltpu.sync_copy(ones_v, dacc_sh.at[dst_v.at[n - 1]],
                                    add=True)
                _swait(1)
            else:
                if with_deg:
                    _dwait(1)
                _swait(0)

        off = 0
        for n in PHASES:
            _phase(off, n)
            off += n

        plsc.subcore_barrier()

        pltpu.sync_copy(acc_sh.at[pl.ds(row0, ROWS_PER_TEC)],
                        agg_hbm.at[c, pl.ds(row0, ROWS_PER_TEC)])
        if with_deg:
            pltpu.sync_copy(dacc_sh.at[pl.ds(row0, ROWS_PER_TEC)],
                            deg_hbm.at[c, pl.ds(row0, ROWS_PER_TEC)])

    return agg_kernel(src2d, dst2d, table)


def _tc_dense(agg, deg, xin, W_l, b, W_r, relu):
    B = 512

    def body(agg_ref, deg_ref, x_ref, wl_ref, wr_ref, b_ref, o_ref):
        ssum = agg_ref[0] + agg_ref[1]
        dsum = deg_ref[0] + deg_ref[1]
        mean = ssum / jnp.maximum(dsum, 1.0)
        acc = (jnp.dot(mean, wl_ref[...], preferred_element_type=jnp.float32)
               + jnp.dot(x_ref[...], wr_ref[...], preferred_element_type=jnp.float32)
               + b_ref[...])
        o_ref[...] = jnp.maximum(acc, 0.0) if relu else acc

    return pl.pallas_call(
        body,
        grid=(N_PAD // B,),
        in_specs=[
            pl.BlockSpec((2, B, D), lambda i: (0, i, 0)),
            pl.BlockSpec((2, B, 1), lambda i: (0, i, 0)),
            pl.BlockSpec((B, D), lambda i: (i, 0)),
            pl.BlockSpec((D, D), lambda i: (0, 0)),
            pl.BlockSpec((D, D), lambda i: (0, 0)),
            pl.BlockSpec((1, D), lambda i: (0, 0)),
        ],
        out_specs=pl.BlockSpec((B, D), lambda i: (i, 0)),
        out_shape=jax.ShapeDtypeStruct((N_PAD, D), jnp.float32),
    )(agg, deg, xin, W_l, W_r, b)


def kernel(x, edge_index, W1_l, b1, W1_r, W2_l, b2, W2_r):
    src = edge_index[0].astype(jnp.int32).reshape(N_TECS, CHUNKS_PER_TEC, CHUNK)
    dst = edge_index[1].astype(jnp.int32).reshape(N_TECS, CHUNKS_PER_TEC, CHUNK)
    x_pad = jnp.pad(x, ((0, N_PAD - N_NODES), (0, 0)))
    b1r = b1.reshape(1, D)
    b2r = b2.reshape(1, D)

    agg1, deg1 = _sc_aggregate(src, dst, x_pad, with_deg=True)
    deg1 = deg1.reshape(2, N_PAD, 1)
    h = _tc_dense(agg1, deg1, x_pad, W1_l, b1r, W1_r, relu=True)
    (agg2,) = _sc_aggregate(src, dst, h, with_deg=False)
    out = _tc_dense(agg2, deg1, h, W2_l, b2r, W2_r, relu=False)
    return out[:N_NODES]
